# Initial kernel scaffold; baseline (speedup 1.0000x reference)
import jax
import jax.numpy as jnp
from jax.experimental import pallas as pl


def kernel(input_path, observe_path, Wmu, S):
    raise NotImplementedError("write your pallas kernel here")


# speedup vs baseline: 19.8025x; 19.8025x over previous
import numpy as np
import jax
import jax.numpy as jnp
from jax.experimental import pallas as pl
from jax.experimental.pallas import tpu as pltpu

_L = 2
_T = 50
_N = 16384
_DX = 8
_NSUB = 2 ** _L
_HL = np.float32(2.0 ** (-_L))
_SQ = np.float32(np.sqrt(2.0 ** (-_L)))
_R = 128
_C = 128
_NBLK = 8
_BLK = _N // _NBLK
_BROWS = _BLK // _C

_INTERPRET = False


def _step_kernel(ip_ref, obs_ref, wmu_ref, s_ref, d1_ref, dw_ref, dice_ref,
                 out_ref, c_ref, gn_ref, ph_ref, mh_ref):
    t = pl.program_id(0)

    @pl.when(t == 0)
    def _():
        for d in range(_DX):
            c_ref[d] = jnp.zeros((_R, _C), jnp.float32)
        gn_ref[...] = jnp.zeros((_R, _C), jnp.float32)

    xs = [c_ref[d] for d in range(_DX)]
    for k in range(_NSUB):
        th = []
        for d in range(_DX):
            acc = xs[0] * wmu_ref[0, d]
            for e in range(1, _DX):
                acc = acc + xs[e] * wmu_ref[e, d]
            th.append(jnp.tanh(acc))
        new_xs = []
        for d in range(_DX):
            dws = dw_ref[0, k, 0] * s_ref[d, 0]
            for e in range(1, _DX):
                dws = dws + dw_ref[0, k, e] * s_ref[d, e]
            new_xs.append(xs[d] + th[d] * _HL + dws)
        xs = new_xs

    rowi = jax.lax.broadcasted_iota(jnp.int32, (_R, _C), 0)
    lanei = jax.lax.broadcasted_iota(jnp.int32, (_R, _C), 1)
    last = (rowi == _R - 1) & (lanei == _C - 1)
    xs = [jnp.where(last, ip_ref[t + 1, d], xs[d]) for d in range(_DX)]
    for d in range(_DX):
        ph_ref[t + 1, d] = xs[d]
    d0 = xs[0] - obs_ref[t + 1, 0]
    ll = d0 * d0
    for d in range(1, _DX):
        diff = xs[d] - obs_ref[t + 1, d]
        ll = ll + diff * diff
    gnew = -0.5 * ll + gn_ref[...]
    m = jnp.max(gnew)
    what = jnp.exp(gnew - m)
    sw = jnp.sum(what)
    wn = what / sw
    ess = 1.0 / jnp.sum(wn * wn)
    do_res = ess <= _N / 2.0
    ci = jax.lax.broadcasted_iota(jnp.int32, (_C, _C), 0)
    cj = jax.lax.broadcasted_iota(jnp.int32, (_C, _C), 1)
    ut = (ci <= cj).astype(jnp.float32)
    lts = (cj < ci).astype(jnp.float32)
    rowcum = jnp.dot(wn, ut, preferred_element_type=jnp.float32)
    rowtot = rowcum[:, _C - 1:_C]
    prefix = jnp.dot(lts, rowtot, preferred_element_type=jnp.float32)
    bins = rowcum + prefix
    rl_row = jnp.transpose(rowtot + prefix)

    xcat = jnp.concatenate(xs, axis=1)

    def _spread(v2d):
        v3 = jax.lax.broadcast_in_dim(v2d, (_BROWS, _C, _C), (0, 1))
        return v3.reshape(_BLK, _C)

    @pl.when(jnp.logical_and(do_res, t < _T - 1))
    def _():
        gn_ref[...] = jnp.zeros((_R, _C), jnp.float32)
        dice = dice_ref[0]
        lanes = jax.lax.broadcasted_iota(jnp.int32, (_BLK, _C), 1)
        rl_b = jax.lax.broadcast_in_dim(rl_row, (_BLK, _C), (0, 1))
        for b in range(_NBLK):
            d16 = dice[b * _BROWS:(b + 1) * _BROWS, :]
            dsp = _spread(d16)
            rcount = jnp.sum((rl_b <= dsp).astype(jnp.int32), axis=1)
            rc2 = rcount.reshape(_BROWS, _C)
            ohg = (_spread(rc2) == lanes).astype(jnp.float32)
            rowsb = jnp.dot(ohg, bins, preferred_element_type=jnp.float32)
            fine = jnp.sum((rowsb <= dsp).astype(jnp.int32), axis=1)
            idx2 = jnp.minimum(rc2 * _C + fine.reshape(_BROWS, _C), _N - 1)
            if b == _NBLK - 1:
                ri16 = jax.lax.broadcasted_iota(jnp.int32, (_BROWS, _C), 0)
                li16 = jax.lax.broadcasted_iota(jnp.int32, (_BROWS, _C), 1)
                idx2 = jnp.where((ri16 == _BROWS - 1) & (li16 == _C - 1),
                                 _N - 1, idx2)
            mh_ref[t, b * _BROWS:(b + 1) * _BROWS, :] = idx2
            rm = idx2 // _C
            cm = jnp.remainder(idx2, _C)
            ohr = (_spread(rm) == lanes).astype(jnp.float32)
            ohc = (_spread(cm) == lanes).astype(jnp.float32)
            rows = jnp.dot(ohr, xcat, preferred_element_type=jnp.float32)
            for d in range(_DX):
                seld = jnp.sum(rows[:, d * _C:(d + 1) * _C] * ohc, axis=1)
                c_ref[d, b * _BROWS:(b + 1) * _BROWS, :] = seld.reshape(_BROWS, _C)

    @pl.when(jnp.logical_and(jnp.logical_not(do_res), t < _T - 1))
    def _():
        gn_ref[...] = gnew
        for d in range(_DX):
            c_ref[d] = xs[d]
        mh_ref[t] = rowi * _C + lanei

    @pl.when(t == _T - 1)
    def _():
        d1 = d1_ref[0, 0]
        idx1 = jnp.sum((bins <= d1).astype(jnp.int32))
        jcur = jnp.minimum(idx1, _N - 1)
        fi = rowi * _C + lanei
        for d in range(_DX):
            out_ref[_T, d] = jnp.sum(jnp.where(fi == jcur, xs[d], 0.0))
        for s in range(_T - 1, 0, -1):
            jcur = jnp.sum(jnp.where(fi == jcur, mh_ref[s - 1], 0))
            for d in range(_DX):
                out_ref[s, d] = jnp.sum(jnp.where(fi == jcur, ph_ref[s, d], 0.0))
        for d in range(_DX):
            out_ref[0, d] = jnp.where(jcur == _N - 1, ip_ref[0, d],
                                      jnp.float32(0.0))


def kernel(input_path, observe_path, Wmu, S):
    key = jax.random.key(42)
    dw_ctr = (5 * jnp.arange(_T)[:, None]
              + jnp.arange(1, _NSUB + 1)[None, :]).reshape(-1)
    dw_keys = jax.vmap(lambda c: jax.random.fold_in(key, c))(dw_ctr)
    dwn = jax.vmap(
        lambda k: jax.random.normal(k, (_N, _DX, 1), dtype=jnp.float32)
    )(dw_keys)
    dw = (dwn[..., 0] * _SQ).transpose(0, 2, 1).reshape(_T, _NSUB, _DX, _R, _C)
    dice_ctr = 5 * jnp.arange(_T) + _NSUB + 1
    dice_keys = jax.vmap(lambda c: jax.random.fold_in(key, c))(dice_ctr)
    dice = jax.vmap(
        lambda k: jax.random.uniform(k, (_N,), dtype=jnp.float32)
    )(dice_keys).reshape(_T, _R, _C)
    d1 = jax.random.uniform(jax.random.fold_in(key, 5 * _T + 1), (1,),
                            dtype=jnp.float32).reshape(1, 1)

    out = pl.pallas_call(
        _step_kernel,
        grid=(_T,),
        in_specs=[
            pl.BlockSpec(memory_space=pltpu.SMEM),
            pl.BlockSpec(memory_space=pltpu.SMEM),
            pl.BlockSpec(memory_space=pltpu.SMEM),
            pl.BlockSpec(memory_space=pltpu.SMEM),
            pl.BlockSpec(memory_space=pltpu.SMEM),
            pl.BlockSpec((1, _NSUB, _DX, _R, _C), lambda t: (t, 0, 0, 0, 0)),
            pl.BlockSpec((1, _R, _C), lambda t: (t, 0, 0)),
        ],
        out_specs=pl.BlockSpec(memory_space=pltpu.SMEM),
        out_shape=jax.ShapeDtypeStruct((_T + 1, _DX), jnp.float32),
        scratch_shapes=[
            pltpu.VMEM((_DX, _R, _C), jnp.float32),
            pltpu.VMEM((_R, _C), jnp.float32),
            pltpu.VMEM((_T + 1, _DX, _R, _C), jnp.float32),
            pltpu.VMEM((_T, _R, _C), jnp.int32),
        ],
        interpret=_INTERPRET,
    )(input_path, observe_path, Wmu, S, d1, dw, dice)
    return out
